# baseline (speedup 1.0000x reference)
import functools

import jax
import jax.numpy as jnp
from jax import lax
from jax.experimental import pallas as pl
from jax.experimental.pallas import tpu as pltpu
from jax.experimental.pallas import tpu_sc as plsc

BATCH = 4
TOTAL_SEQ = 2048
D_MODEL = 1024
N_POS = 68
ROWS = BATCH * N_POS
ROWS_PER_WORKER = 16
N_CHUNKS = ROWS // ROWS_PER_WORKER


def _flat_positions():
  pos = []
  for b in range(BATCH):
    for j in range(4):
      for k in range(17):
        pos.append(b * TOTAL_SEQ + j * 512 + k * 30)
  return jnp.asarray(pos, dtype=jnp.int32)


def _sc_gather(table, idx):
  mesh = plsc.VectorSubcoreMesh(
      core_axis_name="c", subcore_axis_name="s", num_cores=1
  )

  @functools.partial(
      pl.kernel,
      mesh=mesh,
      out_type=jax.ShapeDtypeStruct((ROWS, D_MODEL), jnp.float32),
      scratch_types=[
          pltpu.VMEM((ROWS_PER_WORKER,), jnp.int32),
          pltpu.VMEM((ROWS_PER_WORKER, D_MODEL), jnp.float32),
          pltpu.SemaphoreType.DMA,
      ],
  )
  def k(table_hbm, idx_hbm, out_hbm, idx_v, rows_v, sem):
    wid = lax.axis_index("s")

    def do_chunk(chunk):
      base = chunk * ROWS_PER_WORKER
      pltpu.sync_copy(idx_hbm.at[pl.ds(base, ROWS_PER_WORKER)], idx_v)
      pltpu.async_copy(table_hbm.at[idx_v], rows_v, sem).wait()
      pltpu.sync_copy(rows_v, out_hbm.at[pl.ds(base, ROWS_PER_WORKER)])

    do_chunk(wid)

    @pl.when(wid == 0)
    def _():
      do_chunk(16)

  return k(table, idx)


def kernel(hidden_states):
  table = hidden_states.reshape(BATCH * TOTAL_SEQ, D_MODEL)
  out = _sc_gather(table, _flat_positions())
  return out.reshape(BATCH, N_POS, D_MODEL)


# speedup vs baseline: 1.0297x; 1.0297x over previous
import functools

import jax
import jax.numpy as jnp
from jax import lax
from jax.experimental import pallas as pl
from jax.experimental.pallas import tpu as pltpu
from jax.experimental.pallas import tpu_sc as plsc

BATCH = 4
TOTAL_SEQ = 2048
D_MODEL = 1024
N_POS = 68
ROWS = BATCH * N_POS
CHUNK = 8
N_CHUNKS = ROWS // CHUNK
N_WORKERS = 32


def _vbcast(x):
  return lax.broadcast(x, (16,))


def _sc_gather(table):
  mesh = plsc.VectorSubcoreMesh(core_axis_name="c", subcore_axis_name="s")

  @functools.partial(
      pl.kernel,
      mesh=mesh,
      out_type=jax.ShapeDtypeStruct((ROWS, D_MODEL), jnp.float32),
      scratch_types=[
          pltpu.VMEM((16,), jnp.int32),
          pltpu.VMEM((CHUNK, D_MODEL), jnp.float32),
          pltpu.SemaphoreType.DMA,
      ],
  )
  def k(table_hbm, out_hbm, idx_v, rows_v, sem):
    wid = lax.axis_index("s") * 2 + lax.axis_index("c")

    def do_chunk(chunk):
      r = _vbcast(chunk * CHUNK) + lax.iota(jnp.int32, 16)
      bj = lax.div(r, _vbcast(jnp.int32(17)))
      kk = r - bj * _vbcast(jnp.int32(17))
      b = lax.div(bj, _vbcast(jnp.int32(4)))
      j = bj - b * _vbcast(jnp.int32(4))
      idx_v[...] = (
          b * _vbcast(jnp.int32(TOTAL_SEQ))
          + j * _vbcast(jnp.int32(512))
          + kk * _vbcast(jnp.int32(30))
      )
      pltpu.async_copy(
          table_hbm.at[idx_v.at[pl.ds(0, CHUNK)]], rows_v, sem
      ).wait()
      pltpu.sync_copy(rows_v, out_hbm.at[pl.ds(chunk * CHUNK, CHUNK)])

    do_chunk(wid)

    @pl.when(wid < N_CHUNKS - N_WORKERS)
    def _():
      do_chunk(wid + N_WORKERS)

  return k(table)


def kernel(hidden_states):
  table = hidden_states.reshape(BATCH * TOTAL_SEQ, D_MODEL)
  out = _sc_gather(table)
  return out.reshape(BATCH, N_POS, D_MODEL)
